# baseline (speedup 1.0000x reference)
import functools

import jax
import jax.numpy as jnp
from jax.experimental import pallas as pl
from jax.experimental.pallas import tpu as pltpu

NEG_INF = float("-inf")


def _fused_body(labels_ref, h_ref,
                wc1_ref, wf01_ref, wf11_ref,
                bc1_ref, bf01_ref, bf11_ref,
                wc2_ref, wf02_ref, wf12_ref,
                bc2_ref, bf02_ref, bf12_ref,
                coarse_out, fine_out, flat_out,
                acc_c, acc_0, acc_1, wbf,
                *, n_h, bt):
    hstep = pl.program_id(0)
    b = pl.program_id(1)
    rows = pl.ds(b * bt, bt)
    hh = h_ref[...].astype(jnp.bfloat16)

    @pl.when(b == 0)
    def _():
        wbf[0] = wc1_ref[...].astype(jnp.bfloat16)
        wbf[1] = wf01_ref[...].astype(jnp.bfloat16)
        wbf[2] = wf11_ref[...].astype(jnp.bfloat16)

    def head_partial(w1_idx, b1_ref, w2_ref):
        z = jax.lax.dot_general(
            hh, wbf[w1_idx], (((1,), (1,)), ((), ())),
            preferred_element_type=jnp.float32)
        z = z + b1_ref[...]
        z = (z * 0.5 * (1.0 + jax.lax.erf(z * 0.7071067811865476))
             ).astype(jnp.bfloat16)
        return jax.lax.dot_general(
            z, w2_ref[...], (((1,), (1,)), ((), ())),
            preferred_element_type=jnp.float32)

    pc = head_partial(0, bc1_ref, wc2_ref)
    p0 = head_partial(1, bf01_ref, wf02_ref)
    p1 = head_partial(2, bf11_ref, wf12_ref)

    @pl.when(hstep == 0)
    def _():
        acc_c[rows, :] = pc
        acc_0[rows, :] = p0
        acc_1[rows, :] = p1

    @pl.when(hstep != 0)
    def _():
        acc_c[rows, :] += pc
        acc_0[rows, :] += p0
        acc_1[rows, :] += p1

    @pl.when(hstep == n_h - 1)
    def _():
        coarse_out[...] = acc_c[rows, :] + bc2_ref[...]
        l0 = acc_0[rows, :] + bf02_ref[...]
        l1 = acc_1[rows, :] + bf12_ref[...]
        nf0 = l0.shape[1]
        nf1 = l1.shape[1]
        mask = labels_ref[...] == 0
        neg = jnp.float32(NEG_INF)
        pad0 = jnp.concatenate(
            [l0, jnp.full((l0.shape[0], nf1 - nf0), neg, jnp.float32)], axis=1)
        fine_out[...] = jnp.where(mask, pad0, l1)
        flat_out[...] = jnp.concatenate(
            [jnp.where(mask, l0, neg), jnp.where(mask, neg, l1)], axis=1)


def kernel(h, coarse_labels, Wc1, bc1, Wc2, bc2,
           Wf0_1, bf0_1, Wf0_2, bf0_2, Wf1_1, bf1_1, Wf1_2, bf1_2):
    B, IN = h.shape
    H = Wc1.shape[0]
    NC = Wc2.shape[0]
    NF0 = Wf0_2.shape[0]
    NF1 = Wf1_2.shape[0]
    bt = min(256, B)
    hc = min(256, H)
    n_b = B // bt
    n_h = H // hc

    bf = jnp.bfloat16
    w1s = [Wc1, Wf0_1, Wf1_1]
    b1s = [bc1.reshape(1, H), bf0_1.reshape(1, H), bf1_1.reshape(1, H)]
    w2s = [Wc2.astype(bf), Wf0_2.astype(bf), Wf1_2.astype(bf)]
    b2s = [bc2.reshape(1, NC), bf0_2.reshape(1, NF0), bf1_2.reshape(1, NF1)]
    labels2 = coarse_labels.reshape(B, 1)

    w1_spec = pl.BlockSpec((hc, IN), lambda hs, b: (hs, 0))
    b1_spec = pl.BlockSpec((1, hc), lambda hs, b: (0, hs))

    def w2_spec(n):
        return pl.BlockSpec((n, hc), lambda hs, b: (0, hs))

    def b2_spec(n):
        return pl.BlockSpec((1, n), lambda hs, b: (0, 0))

    def out_spec(n):
        return pl.BlockSpec((bt, n), lambda hs, b: (b, 0))

    in_specs = [
            pl.BlockSpec((bt, 1), lambda hs, b: (b, 0)),
            pl.BlockSpec((bt, IN), lambda hs, b: (b, 0)),
            w1_spec, w1_spec, w1_spec,
            b1_spec, b1_spec, b1_spec,
            w2_spec(NC), w2_spec(NF0), w2_spec(NF1),
            b2_spec(NC), b2_spec(NF0), b2_spec(NF1),
    ]
    out_specs = [out_spec(NC), out_spec(NF1), out_spec(NF0 + NF1)]

    out_shapes = [
        jax.ShapeDtypeStruct((B, NC), jnp.float32),
        jax.ShapeDtypeStruct((B, NF1), jnp.float32),
        jax.ShapeDtypeStruct((B, NF0 + NF1), jnp.float32),
    ]

    coarse, fine, flat = pl.pallas_call(
        functools.partial(_fused_body, n_h=n_h, bt=bt),
        grid=(n_h, n_b),
        in_specs=in_specs,
        out_specs=out_specs,
        out_shape=out_shapes,
        scratch_shapes=[
            pltpu.VMEM((B, NC), jnp.float32),
            pltpu.VMEM((B, NF0), jnp.float32),
            pltpu.VMEM((B, NF1), jnp.float32),
            pltpu.VMEM((3, hc, IN), jnp.bfloat16),
        ],
    )(labels2, h, *w1s, *b1s, *w2s, *b2s)
    return (coarse, fine, flat)


# speedup vs baseline: 1.6125x; 1.6125x over previous
import functools

import jax
import jax.numpy as jnp
from jax.experimental import pallas as pl
from jax.experimental.pallas import tpu as pltpu

NEG_INF = float("-inf")


def _fused_body(labels_ref, h_ref,
                wc1_ref, wf01_ref, wf11_ref,
                bc1_ref, bf01_ref, bf11_ref,
                wc2_ref, wf02_ref, wf12_ref,
                bc2_ref, bf02_ref, bf12_ref,
                coarse_out, fine_out, flat_out,
                acc_c, acc_0, acc_1,
                *, n_h):
    hstep = pl.program_id(1)
    hh = h_ref[...].astype(jnp.bfloat16)

    def head_partial(w1_ref, b1_ref, w2_ref):
        z = jax.lax.dot_general(
            hh, w1_ref[...], (((1,), (1,)), ((), ())),
            preferred_element_type=jnp.float32)
        z = z + b1_ref[...]
        z = (z * 0.5 * (1.0 + jax.lax.erf(z * 0.7071067811865476))
             ).astype(jnp.bfloat16)
        return jax.lax.dot_general(
            z, w2_ref[...], (((1,), (1,)), ((), ())),
            preferred_element_type=jnp.float32)

    pc = head_partial(wc1_ref, bc1_ref, wc2_ref)
    p0 = head_partial(wf01_ref, bf01_ref, wf02_ref)
    p1 = head_partial(wf11_ref, bf11_ref, wf12_ref)

    @pl.when(hstep == 0)
    def _():
        acc_c[...] = pc
        acc_0[...] = p0
        acc_1[...] = p1

    @pl.when(hstep != 0)
    def _():
        acc_c[...] += pc
        acc_0[...] += p0
        acc_1[...] += p1

    @pl.when(hstep == n_h - 1)
    def _():
        coarse_out[...] = acc_c[...] + bc2_ref[...]
        l0 = acc_0[...] + bf02_ref[...]
        l1 = acc_1[...] + bf12_ref[...]
        nf0 = l0.shape[1]
        nf1 = l1.shape[1]
        mask = labels_ref[...] == 0
        neg = jnp.float32(NEG_INF)
        pad0 = jnp.concatenate(
            [l0, jnp.full((l0.shape[0], nf1 - nf0), neg, jnp.float32)], axis=1)
        fine_out[...] = jnp.where(mask, pad0, l1)
        flat_out[...] = jnp.concatenate(
            [jnp.where(mask, l0, neg), jnp.where(mask, neg, l1)], axis=1)


def kernel(h, coarse_labels, Wc1, bc1, Wc2, bc2,
           Wf0_1, bf0_1, Wf0_2, bf0_2, Wf1_1, bf1_1, Wf1_2, bf1_2):
    B, IN = h.shape
    H = Wc1.shape[0]
    NC = Wc2.shape[0]
    NF0 = Wf0_2.shape[0]
    NF1 = Wf1_2.shape[0]
    bt = min(512, B)
    hc = min(512, H)
    n_b = B // bt
    n_h = H // hc

    bf = jnp.bfloat16
    w1s = [Wc1.astype(bf), Wf0_1.astype(bf), Wf1_1.astype(bf)]
    b1s = [bc1.reshape(1, H), bf0_1.reshape(1, H), bf1_1.reshape(1, H)]
    w2s = [Wc2.astype(bf), Wf0_2.astype(bf), Wf1_2.astype(bf)]
    b2s = [bc2.reshape(1, NC), bf0_2.reshape(1, NF0), bf1_2.reshape(1, NF1)]
    labels2 = coarse_labels.reshape(B, 1)

    w1_spec = pl.BlockSpec((hc, IN), lambda b, hs: (hs, 0))
    b1_spec = pl.BlockSpec((1, hc), lambda b, hs: (0, hs))

    def w2_spec(n):
        return pl.BlockSpec((n, hc), lambda b, hs: (0, hs))

    def b2_spec(n):
        return pl.BlockSpec((1, n), lambda b, hs: (0, 0))

    def out_spec(n):
        return pl.BlockSpec((bt, n), lambda b, hs: (b, 0))

    in_specs = [
            pl.BlockSpec((bt, 1), lambda b, hs: (b, 0)),
            pl.BlockSpec((bt, IN), lambda b, hs: (b, 0)),
            w1_spec, w1_spec, w1_spec,
            b1_spec, b1_spec, b1_spec,
            w2_spec(NC), w2_spec(NF0), w2_spec(NF1),
            b2_spec(NC), b2_spec(NF0), b2_spec(NF1),
    ]
    out_specs = [out_spec(NC), out_spec(NF1), out_spec(NF0 + NF1)]

    out_shapes = [
        jax.ShapeDtypeStruct((B, NC), jnp.float32),
        jax.ShapeDtypeStruct((B, NF1), jnp.float32),
        jax.ShapeDtypeStruct((B, NF0 + NF1), jnp.float32),
    ]

    coarse, fine, flat = pl.pallas_call(
        functools.partial(_fused_body, n_h=n_h),
        grid=(n_b, n_h),
        in_specs=in_specs,
        out_specs=out_specs,
        out_shape=out_shapes,
        scratch_shapes=[
            pltpu.VMEM((bt, NC), jnp.float32),
            pltpu.VMEM((bt, NF0), jnp.float32),
            pltpu.VMEM((bt, NF1), jnp.float32),
        ],
    )(labels2, h, *w1s, *b1s, *w2s, *b2s)
    return (coarse, fine, flat)


# speedup vs baseline: 1.6332x; 1.0128x over previous
import functools

import jax
import jax.numpy as jnp
from jax.experimental import pallas as pl
from jax.experimental.pallas import tpu as pltpu

NEG_INF = float("-inf")


def _fused_body(labels_ref, h_ref,
                wc1_ref, wf01_ref, wf11_ref,
                bc1_ref, bf01_ref, bf11_ref,
                wc2_ref, wf02_ref, wf12_ref,
                bc2_ref, bf02_ref, bf12_ref,
                coarse_out, fine_out, flat_out,
                acc_c, acc_0, acc_1,
                *, n_h):
    hstep = pl.program_id(1)
    hh = h_ref[...].astype(jnp.bfloat16)

    def l1(w1_ref, b1_ref):
        z = jax.lax.dot_general(
            hh, w1_ref[...], (((1,), (1,)), ((), ())),
            preferred_element_type=jnp.float32)
        return z + b1_ref[...]

    def act(z):
        return (z * 0.5 * (1.0 + jax.lax.erf(z * 0.7071067811865476))
                ).astype(jnp.bfloat16)

    def l2(z, w2_ref):
        return jax.lax.dot_general(
            z, w2_ref[...], (((1,), (1,)), ((), ())),
            preferred_element_type=jnp.float32)

    zc = l1(wc1_ref, bc1_ref)
    z0 = l1(wf01_ref, bf01_ref)
    z1 = l1(wf11_ref, bf11_ref)
    gc, g0, g1 = act(zc), act(z0), act(z1)
    pc = l2(gc, wc2_ref)
    p0 = l2(g0, wf02_ref)
    p1 = l2(g1, wf12_ref)

    @pl.when(hstep == 0)
    def _():
        acc_c[...] = pc
        acc_0[...] = p0
        acc_1[...] = p1

    @pl.when(hstep != 0)
    def _():
        acc_c[...] += pc
        acc_0[...] += p0
        acc_1[...] += p1

    @pl.when(hstep == n_h - 1)
    def _():
        coarse_out[...] = acc_c[...] + bc2_ref[...]
        l0 = acc_0[...] + bf02_ref[...]
        l1 = acc_1[...] + bf12_ref[...]
        nf0 = l0.shape[1]
        nf1 = l1.shape[1]
        mask = labels_ref[...] == 0
        neg = jnp.float32(NEG_INF)
        pad0 = jnp.concatenate(
            [l0, jnp.full((l0.shape[0], nf1 - nf0), neg, jnp.float32)], axis=1)
        fine_out[...] = jnp.where(mask, pad0, l1)
        flat_out[...] = jnp.concatenate(
            [jnp.where(mask, l0, neg), jnp.where(mask, neg, l1)], axis=1)


def kernel(h, coarse_labels, Wc1, bc1, Wc2, bc2,
           Wf0_1, bf0_1, Wf0_2, bf0_2, Wf1_1, bf1_1, Wf1_2, bf1_2):
    B, IN = h.shape
    H = Wc1.shape[0]
    NC = Wc2.shape[0]
    NF0 = Wf0_2.shape[0]
    NF1 = Wf1_2.shape[0]
    bt = min(512, B)
    hc = min(512, H)
    n_b = B // bt
    n_h = H // hc

    bf = jnp.bfloat16
    w1s = [Wc1.astype(bf), Wf0_1.astype(bf), Wf1_1.astype(bf)]
    b1s = [bc1.reshape(1, H), bf0_1.reshape(1, H), bf1_1.reshape(1, H)]
    w2s = [Wc2.astype(bf), Wf0_2.astype(bf), Wf1_2.astype(bf)]
    b2s = [bc2.reshape(1, NC), bf0_2.reshape(1, NF0), bf1_2.reshape(1, NF1)]
    labels2 = coarse_labels.reshape(B, 1)

    w1_spec = pl.BlockSpec((hc, IN), lambda b, hs: (hs, 0))
    b1_spec = pl.BlockSpec((1, hc), lambda b, hs: (0, hs))

    def w2_spec(n):
        return pl.BlockSpec((n, hc), lambda b, hs: (0, hs))

    def b2_spec(n):
        return pl.BlockSpec((1, n), lambda b, hs: (0, 0))

    def out_spec(n):
        return pl.BlockSpec((bt, n), lambda b, hs: (b, 0))

    in_specs = [
            pl.BlockSpec((bt, 1), lambda b, hs: (b, 0)),
            pl.BlockSpec((bt, IN), lambda b, hs: (b, 0)),
            w1_spec, w1_spec, w1_spec,
            b1_spec, b1_spec, b1_spec,
            w2_spec(NC), w2_spec(NF0), w2_spec(NF1),
            b2_spec(NC), b2_spec(NF0), b2_spec(NF1),
    ]
    out_specs = [out_spec(NC), out_spec(NF1), out_spec(NF0 + NF1)]

    out_shapes = [
        jax.ShapeDtypeStruct((B, NC), jnp.float32),
        jax.ShapeDtypeStruct((B, NF1), jnp.float32),
        jax.ShapeDtypeStruct((B, NF0 + NF1), jnp.float32),
    ]

    coarse, fine, flat = pl.pallas_call(
        functools.partial(_fused_body, n_h=n_h),
        grid=(n_b, n_h),
        in_specs=in_specs,
        out_specs=out_specs,
        out_shape=out_shapes,
        scratch_shapes=[
            pltpu.VMEM((bt, NC), jnp.float32),
            pltpu.VMEM((bt, NF0), jnp.float32),
            pltpu.VMEM((bt, NF1), jnp.float32),
        ],
    )(labels2, h, *w1s, *b1s, *w2s, *b2s)
    return (coarse, fine, flat)
